# Initial kernel scaffold; baseline (speedup 1.0000x reference)
import jax
import jax.numpy as jnp
from jax.experimental import pallas as pl


def kernel(x, edge_index, W):
    raise NotImplementedError("write your pallas kernel here")


# speedup vs baseline: 4.9476x; 4.9476x over previous
import functools

import jax
import jax.numpy as jnp
from jax import lax
from jax.experimental import pallas as pl
from jax.experimental.pallas import tpu as pltpu
from jax.experimental.pallas import tpu_sc as plsc

N_NODES = 10000
N_EDGES = 320000
D_FEAT = 128

NC = 2
NS = 16
NW = NC * NS
LANES = 16

CHUNK = 128
K_CHUNKS = -(-N_EDGES // (NW * CHUNK))
E_PAD = NW * K_CHUNKS * CHUNK
ACC_ROWS = 10240
STRIPE = ACC_ROWS // NS
STRIPE_BLKS = STRIPE // CHUNK


def _sc_segment_sum(x, src3, dst3):
    mesh = plsc.VectorSubcoreMesh(core_axis_name="c", subcore_axis_name="s")

    @functools.partial(
        pl.kernel,
        mesh=mesh,
        out_type=jax.ShapeDtypeStruct((NC, ACC_ROWS, D_FEAT), jnp.float32),
        scratch_types=[
            pltpu.VMEM((K_CHUNKS, CHUNK), jnp.int32),
            pltpu.VMEM((K_CHUNKS, CHUNK), jnp.int32),
            pltpu.VMEM((CHUNK, D_FEAT), jnp.float32),
            pltpu.VMEM_SHARED((ACC_ROWS, D_FEAT), jnp.float32),
            pltpu.SemaphoreType.DMA,
        ],
    )
    def k(x_hbm, src_hbm, dst_hbm, out_hbm, sidx, didx, rows, acc, sem):
        c = lax.axis_index("c")
        s = lax.axis_index("s")
        wid = s * NC + c

        @pl.loop(0, CHUNK)
        def _(r):
            @pl.loop(0, D_FEAT, step=LANES)
            def _(col):
                rows.at[pl.ds(r, 1), pl.ds(col, LANES)][...] = jnp.zeros(
                    (1, LANES), jnp.float32)

        @pl.loop(0, STRIPE_BLKS)
        def _(b):
            pltpu.sync_copy(rows, acc.at[pl.ds(s * STRIPE + b * CHUNK, CHUNK)])

        plsc.subcore_barrier()

        pltpu.sync_copy(src_hbm.at[wid], sidx)
        pltpu.sync_copy(dst_hbm.at[wid], didx)

        @pl.loop(0, K_CHUNKS)
        def _(j):
            pltpu.async_copy(x_hbm.at[sidx.at[j]], rows, sem).wait()
            pltpu.sync_copy(rows, acc.at[didx.at[j]], add=True)

        plsc.subcore_barrier()

        @pl.loop(0, STRIPE_BLKS)
        def _(b):
            base = s * STRIPE + b * CHUNK
            pltpu.sync_copy(acc.at[pl.ds(base, CHUNK)],
                            out_hbm.at[c, pl.ds(base, CHUNK)])

    return k(x, src3, dst3)


def _combine(parts, W):
    blk = 1000

    def body(p_ref, w_ref, o_ref):
        o_ref[...] = (p_ref[0] + p_ref[1]) * w_ref[...]

    return pl.pallas_call(
        body,
        grid=(N_NODES // blk,),
        in_specs=[
            pl.BlockSpec((NC, blk, D_FEAT), lambda i: (0, i, 0)),
            pl.BlockSpec((1, D_FEAT), lambda i: (0, 0)),
        ],
        out_specs=pl.BlockSpec((blk, D_FEAT), lambda i: (i, 0)),
        out_shape=jax.ShapeDtypeStruct((N_NODES, D_FEAT), jnp.float32),
    )(parts, W)


def kernel(x, edge_index, W):
    src = edge_index[0]
    dst = edge_index[1]
    pad = E_PAD - N_EDGES
    src3 = jnp.concatenate([src, jnp.zeros((pad,), jnp.int32)]).reshape(
        NW, K_CHUNKS, CHUNK)
    dst3 = jnp.concatenate([dst, jnp.full((pad,), N_NODES, jnp.int32)]).reshape(
        NW, K_CHUNKS, CHUNK)
    parts = _sc_segment_sum(x, src3, dst3)
    return _combine(parts, W)
